# baseline (speedup 1.0000x reference)
import jax
import jax.numpy as jnp
from jax import lax
from jax.experimental import pallas as pl
from jax.experimental.pallas import tpu as pltpu
from jax.experimental.pallas import tpu_sc as plsc

N_NODES = 10000
N_EDGES = 320000
D_IN = 128
D_HID = 128
D_OUT = 64

NC = 2
NS = 16
NW = NC * NS
EDGES_PER_W = N_EDGES // NW
CHUNK = 128
N_CHUNKS = 80
RING = 5
N_GROUPS = N_CHUNKS // RING
E_PAD = NW * N_CHUNKS * CHUNK
ACC_ROWS = N_NODES + 8
STRIPE_A = 632
STRIPE_B = N_NODES - 15 * STRIPE_A

_SC_PARAMS = pltpu.CompilerParams(use_tc_tiling_on_sc=False)


def _stripe_copy(s, src_at, dst_at):
  r0 = s * STRIPE_A

  @pl.when(s < NS - 1)
  def _():
    pltpu.sync_copy(src_at(pl.ds(r0, STRIPE_A)), dst_at(pl.ds(r0, STRIPE_A)))

  @pl.when(s == NS - 1)
  def _():
    pltpu.sync_copy(src_at(pl.ds(15 * STRIPE_A, STRIPE_B)),
                    dst_at(pl.ds(15 * STRIPE_A, STRIPE_B)))


def _make_propagate(d: int, dtype=jnp.bfloat16):
  mesh = plsc.VectorSubcoreMesh(core_axis_name="c", subcore_axis_name="s")
  n_chunks = N_CHUNKS
  n_groups = N_GROUPS

  def body(table_hbm, src_hbm, dst_hbm, p_hbm, srcv, dstv, rows, acc,
           gsems, ssems):
    c = lax.axis_index("c")
    s = lax.axis_index("s")
    w = c * NS + s

    pltpu.sync_copy(src_hbm.at[pl.ds(w * N_CHUNKS, N_CHUNKS)], srcv)
    pltpu.sync_copy(dst_hbm.at[pl.ds(w * N_CHUNKS, N_CHUNKS)], dstv)
    _stripe_copy(s, lambda sl: table_hbm.at[sl], lambda sl: acc.at[sl])
    plsc.subcore_barrier()

    def fire_gather(i, j):
      return pltpu.async_copy(table_hbm.at[srcv.at[i]], rows[j], gsems[j])

    def fire_scatter(i, j):
      return pltpu.async_copy(rows[j], acc.at[dstv.at[i]], ssems[j], add=True)

    for j in range(RING):
      fire_gather(j, j)

    def group(n, carry):
      for j in range(RING):
        i = n * RING + j
        pltpu.make_async_copy(table_hbm.at[srcv.at[i]], rows[j],
                              gsems[j]).wait()
        fire_scatter(i, j)

        @pl.when(i + RING < n_chunks)
        def _():
          pltpu.make_async_copy(rows[j], acc.at[dstv.at[i]], ssems[j]).wait()
          fire_gather(i + RING, j)
      return carry

    lax.fori_loop(0, n_groups, group, 0)
    for j in range(RING):
      i = n_chunks - RING + j
      pltpu.make_async_copy(rows[j], acc.at[dstv.at[i]], ssems[j]).wait()

    plsc.subcore_barrier()
    _stripe_copy(s, lambda sl: acc.at[sl], lambda sl: p_hbm.at[c, sl])

  return pl.kernel(
      body,
      out_type=jax.ShapeDtypeStruct((NC, N_NODES, d), dtype),
      mesh=mesh,
      compiler_params=_SC_PARAMS,
      scratch_types=[
          pltpu.VMEM((N_CHUNKS, CHUNK), jnp.int32),
          pltpu.VMEM((N_CHUNKS, CHUNK), jnp.int32),
          [pltpu.VMEM((CHUNK, d), dtype) for _ in range(RING)],
          pltpu.VMEM_SHARED((ACC_ROWS, d), dtype),
          [pltpu.SemaphoreType.DMA for _ in range(RING)],
          [pltpu.SemaphoreType.DMA for _ in range(RING)],
      ],
  )


_propagate_128 = _make_propagate(D_HID)
_propagate_64 = _make_propagate(D_OUT)


def _make_degree():
  mesh = plsc.VectorSubcoreMesh(core_axis_name="c", subcore_axis_name="s")
  DD = 16

  def body(ones_hbm, dst_hbm, p_hbm, dstv, ones_v, acc, ssems):
    c = lax.axis_index("c")
    s = lax.axis_index("s")
    w = c * NS + s

    pltpu.sync_copy(dst_hbm.at[pl.ds(w * N_CHUNKS, N_CHUNKS)], dstv)
    pltpu.sync_copy(ones_hbm.at[pl.ds(0, CHUNK)], ones_v)
    _stripe_copy(s, lambda sl: ones_hbm.at[sl], lambda sl: acc.at[sl])
    plsc.subcore_barrier()

    def fire(i, j):
      return pltpu.async_copy(ones_v, acc.at[dstv.at[i]], ssems[j], add=True)

    def group(n, carry):
      for j in range(RING):
        i = n * RING + j

        @pl.when(n > 0)
        def _():
          pltpu.make_async_copy(ones_v, acc.at[dstv.at[i]], ssems[j]).wait()

        fire(i, j)
      return carry

    lax.fori_loop(0, N_GROUPS, group, 0)
    for j in range(RING):
      pltpu.make_async_copy(ones_v, acc.at[dstv.at[0]], ssems[j]).wait()

    plsc.subcore_barrier()
    _stripe_copy(s, lambda sl: acc.at[sl], lambda sl: p_hbm.at[c, sl])

  return pl.kernel(
      body,
      out_type=jax.ShapeDtypeStruct((NC, N_NODES, DD), jnp.float32),
      mesh=mesh,
      compiler_params=_SC_PARAMS,
      scratch_types=[
          pltpu.VMEM((N_CHUNKS, CHUNK), jnp.int32),
          pltpu.VMEM((CHUNK, DD), jnp.float32),
          pltpu.VMEM_SHARED((ACC_ROWS, DD), jnp.float32),
          [pltpu.SemaphoreType.DMA for _ in range(RING)],
      ],
  )


_degree = _make_degree()


BN = 1000
GRID = N_NODES // BN


def _dinv_body(degp_ref, out_ref):
  degp = degp_ref[...]
  deg = degp[0, :, 0:1] + degp[1, :, 0:1] - 1.0
  out_ref[...] = jnp.broadcast_to(lax.rsqrt(jnp.maximum(deg, 1e-12)),
                                  out_ref.shape)


def _mm_first_body(x_ref, w_ref, dinv_ref, out_ref):
  out_ref[...] = (dinv_ref[...] * jnp.dot(x_ref[...], w_ref[...],
                                          preferred_element_type=jnp.float32)
                  ).astype(out_ref.dtype)


def _mm_mid_body(p_ref, hp_ref, b_ref, w_ref, dinv_ref, out_ref):
  dinv = dinv_ref[...]
  pp = p_ref[...].astype(jnp.float32)
  z = dinv * (pp[0] + pp[1] - hp_ref[...].astype(jnp.float32)) + b_ref[...]
  a = jnp.maximum(z, 0.0)
  d_out = out_ref.shape[1]
  out_ref[...] = (dinv[:, :d_out] * jnp.dot(a, w_ref[...],
                                            preferred_element_type=jnp.float32)
                  ).astype(out_ref.dtype)


def _soft_body(p_ref, hp_ref, b_ref, dinv_ref, out_ref):
  dinv = dinv_ref[...][:, :D_OUT]
  pp = p_ref[...].astype(jnp.float32)
  z = dinv * (pp[0] + pp[1] - hp_ref[...].astype(jnp.float32)) + b_ref[...]
  z = z - jnp.max(z, axis=1, keepdims=True)
  ez = jnp.exp(z)
  out_ref[...] = ez / jnp.sum(ez, axis=1, keepdims=True)


def _row_blk(d):
  return pl.BlockSpec((BN, d), lambda i: (i, 0))


def _p_blk(d):
  return pl.BlockSpec((NC, BN, d), lambda i: (0, i, 0))


_DEGP_BLK = pl.BlockSpec((NC, BN, 16), lambda i: (0, i, 0))


def _full_blk(a, b):
  return pl.BlockSpec((a, b), lambda i: (0, 0))


def _dinv_bcast(degp):
  return pl.pallas_call(
      _dinv_body,
      grid=(GRID,),
      in_specs=[_DEGP_BLK],
      out_specs=_row_blk(D_HID),
      out_shape=jax.ShapeDtypeStruct((N_NODES, D_HID), jnp.float32),
  )(degp)


def _mm_first(x, w, dinv):
  return pl.pallas_call(
      _mm_first_body,
      grid=(GRID,),
      in_specs=[_row_blk(D_IN), _full_blk(D_IN, D_HID), _row_blk(D_HID)],
      out_specs=_row_blk(D_HID),
      out_shape=jax.ShapeDtypeStruct((N_NODES, D_HID), jnp.bfloat16),
  )(x, w, dinv)


def _mm_mid(p, hp, b, w, dinv, d_out):
  return pl.pallas_call(
      _mm_mid_body,
      grid=(GRID,),
      in_specs=[_p_blk(D_HID), _row_blk(D_HID), _full_blk(1, D_HID),
                _full_blk(D_HID, d_out), _row_blk(D_HID)],
      out_specs=_row_blk(d_out),
      out_shape=jax.ShapeDtypeStruct((N_NODES, d_out), jnp.bfloat16),
  )(p, hp, b, w, dinv)


def _softmax_out(p, hp, b, dinv):
  return pl.pallas_call(
      _soft_body,
      grid=(GRID,),
      in_specs=[_p_blk(D_OUT), _row_blk(D_OUT), _full_blk(1, D_OUT),
                _row_blk(D_HID)],
      out_specs=_row_blk(D_OUT),
      out_shape=jax.ShapeDtypeStruct((N_NODES, D_OUT), jnp.float32),
  )(p, hp, b, dinv)


def kernel(x, edge_index, W1, b1, W2, b2, W3, b3, W4, b4):
  pad = E_PAD - N_EDGES
  src = jnp.concatenate(
      [edge_index[0].astype(jnp.int32), jnp.zeros((pad,), jnp.int32)]
  ).reshape(E_PAD // CHUNK, CHUNK)
  dst = jnp.concatenate(
      [edge_index[1].astype(jnp.int32),
       jnp.full((pad,), N_NODES, jnp.int32)]
  ).reshape(E_PAD // CHUNK, CHUNK)
  x = x.astype(jnp.float32)

  ones_tab = jnp.ones((N_NODES, 16), jnp.float32)
  degp = _degree(ones_tab, dst)
  dinv = _dinv_bcast(degp)

  h1p = _mm_first(x, W1, dinv)
  p1 = _propagate_128(h1p, src, dst)
  h2p = _mm_mid(p1, h1p, b1.reshape(1, -1), W2, dinv, D_HID)
  p2 = _propagate_128(h2p, src, dst)
  h3p = _mm_mid(p2, h2p, b2.reshape(1, -1), W3, dinv, D_HID)
  p3 = _propagate_128(h3p, src, dst)
  h4p = _mm_mid(p3, h3p, b3.reshape(1, -1), W4, dinv, D_OUT)
  p4 = _propagate_64(h4p, src, dst)
  return _softmax_out(p4, h4p, b4.reshape(1, -1), dinv)


# speedup vs baseline: 2.5524x; 2.5524x over previous
import jax
import jax.numpy as jnp
from jax import lax
from jax.experimental import pallas as pl
from jax.experimental.pallas import tpu as pltpu
from jax.experimental.pallas import tpu_sc as plsc

N_NODES = 10000
N_EDGES = 320000
D_IN = 128
D_HID = 128
D_OUT = 64

NC = 2
NS = 16
NW = NC * NS
EDGES_PER_W = N_EDGES // NW
RING = 5
CHUNK = 80
N_CHUNKS = EDGES_PER_W // CHUNK
N_GROUPS = N_CHUNKS // RING
DCHUNK = 40
DN_CHUNKS = EDGES_PER_W // DCHUNK
DN_GROUPS = DN_CHUNKS // RING
STRIPE_A = 632
STRIPE_B = N_NODES - 15 * STRIPE_A

_SC_PARAMS = pltpu.CompilerParams(use_tc_tiling_on_sc=False)


def _stripe_copy(s, src_at, dst_at):
  r0 = s * STRIPE_A

  @pl.when(s < NS - 1)
  def _():
    pltpu.sync_copy(src_at(pl.ds(r0, STRIPE_A)), dst_at(pl.ds(r0, STRIPE_A)))

  @pl.when(s == NS - 1)
  def _():
    pltpu.sync_copy(src_at(pl.ds(15 * STRIPE_A, STRIPE_B)),
                    dst_at(pl.ds(15 * STRIPE_A, STRIPE_B)))


def _make_propagate(d: int, dtype=jnp.bfloat16):
  mesh = plsc.VectorSubcoreMesh(core_axis_name="c", subcore_axis_name="s")

  def body(table_hbm, src_hbm, dst_hbm, p_hbm, srcv, dstv, rows, acc,
           gsems, ssems):
    c = lax.axis_index("c")
    s = lax.axis_index("s")
    w = c * NS + s

    pltpu.sync_copy(src_hbm.at[w], srcv)
    pltpu.sync_copy(dst_hbm.at[w], dstv)
    _stripe_copy(s, lambda sl: table_hbm.at[sl], lambda sl: acc.at[sl])
    plsc.subcore_barrier()

    def fire_gather(i, j):
      return pltpu.async_copy(table_hbm.at[srcv.at[i]], rows[j], gsems[j])

    def fire_scatter(i, j):
      return pltpu.async_copy(rows[j], acc.at[dstv.at[i]], ssems[j], add=True)

    for j in range(RING):
      fire_gather(j, j)

    def group(n, carry):
      for j in range(RING):
        i = n * RING + j
        pltpu.make_async_copy(table_hbm.at[srcv.at[i]], rows[j],
                              gsems[j]).wait()
        fire_scatter(i, j)

        @pl.when(i + RING < N_CHUNKS)
        def _():
          pltpu.make_async_copy(rows[j], acc.at[dstv.at[i]], ssems[j]).wait()
          fire_gather(i + RING, j)
      return carry

    lax.fori_loop(0, N_GROUPS, group, 0)
    for j in range(RING):
      i = N_CHUNKS - RING + j
      pltpu.make_async_copy(rows[j], acc.at[dstv.at[i]], ssems[j]).wait()

    plsc.subcore_barrier()
    _stripe_copy(s, lambda sl: acc.at[sl], lambda sl: p_hbm.at[c, sl])

  return pl.kernel(
      body,
      out_type=jax.ShapeDtypeStruct((NC, N_NODES, d), dtype),
      mesh=mesh,
      compiler_params=_SC_PARAMS,
      scratch_types=[
          pltpu.VMEM((N_CHUNKS, CHUNK), jnp.int32),
          pltpu.VMEM((N_CHUNKS, CHUNK), jnp.int32),
          [pltpu.VMEM((CHUNK, d), dtype) for _ in range(RING)],
          pltpu.VMEM_SHARED((N_NODES, d), dtype),
          [pltpu.SemaphoreType.DMA for _ in range(RING)],
          [pltpu.SemaphoreType.DMA for _ in range(RING)],
      ],
  )


_propagate_128 = _make_propagate(D_HID)
_propagate_64 = _make_propagate(D_OUT)


def _make_degree():
  mesh = plsc.VectorSubcoreMesh(core_axis_name="c", subcore_axis_name="s")
  DD = 16

  def body(ones_hbm, dst_hbm, p_hbm, dstv, ones_v, acc, ssems):
    c = lax.axis_index("c")
    s = lax.axis_index("s")
    w = c * NS + s

    pltpu.sync_copy(dst_hbm.at[w], dstv)
    pltpu.sync_copy(ones_hbm.at[pl.ds(0, DCHUNK)], ones_v)
    _stripe_copy(s, lambda sl: ones_hbm.at[sl], lambda sl: acc.at[sl])
    plsc.subcore_barrier()

    def fire(i, j):
      return pltpu.async_copy(ones_v, acc.at[dstv.at[i]], ssems[j], add=True)

    def group(n, carry):
      for j in range(RING):
        i = n * RING + j

        @pl.when(n > 0)
        def _():
          pltpu.make_async_copy(ones_v, acc.at[dstv.at[i]], ssems[j]).wait()

        fire(i, j)
      return carry

    lax.fori_loop(0, DN_GROUPS, group, 0)
    for j in range(RING):
      pltpu.make_async_copy(ones_v, acc.at[dstv.at[0]], ssems[j]).wait()

    plsc.subcore_barrier()
    _stripe_copy(s, lambda sl: acc.at[sl], lambda sl: p_hbm.at[c, sl])

  return pl.kernel(
      body,
      out_type=jax.ShapeDtypeStruct((NC, N_NODES, DD), jnp.float32),
      mesh=mesh,
      compiler_params=_SC_PARAMS,
      scratch_types=[
          pltpu.VMEM((DN_CHUNKS, DCHUNK), jnp.int32),
          pltpu.VMEM((DCHUNK, DD), jnp.float32),
          pltpu.VMEM_SHARED((N_NODES, DD), jnp.float32),
          [pltpu.SemaphoreType.DMA for _ in range(RING)],
      ],
  )


_degree = _make_degree()


BN = 1000
GRID = N_NODES // BN


def _dinv_from_degp(degp_blk):
  deg = degp_blk[0, :, 0:1] + degp_blk[1, :, 0:1] - 1.0
  return lax.rsqrt(jnp.maximum(deg, 1e-12))


def _mm_first_body(x_ref, w_ref, degp_ref, out_ref):
  dinv = _dinv_from_degp(degp_ref[...])
  out_ref[...] = (dinv * jnp.dot(x_ref[...], w_ref[...],
                                 preferred_element_type=jnp.float32)
                  ).astype(out_ref.dtype)


def _mm_mid_body(p_ref, hp_ref, b_ref, w_ref, degp_ref, out_ref):
  dinv = _dinv_from_degp(degp_ref[...])
  pp = p_ref[...].astype(jnp.float32)
  z = dinv * (pp[0] + pp[1] - hp_ref[...].astype(jnp.float32)) + b_ref[...]
  a = jnp.maximum(z, 0.0)
  out_ref[...] = (dinv * jnp.dot(a, w_ref[...],
                                 preferred_element_type=jnp.float32)
                  ).astype(out_ref.dtype)


def _soft_body(p_ref, hp_ref, b_ref, degp_ref, out_ref):
  dinv = _dinv_from_degp(degp_ref[...])
  pp = p_ref[...].astype(jnp.float32)
  z = dinv * (pp[0] + pp[1] - hp_ref[...].astype(jnp.float32)) + b_ref[...]
  z = z - jnp.max(z, axis=1, keepdims=True)
  ez = jnp.exp(z)
  out_ref[...] = ez / jnp.sum(ez, axis=1, keepdims=True)


def _row_blk(d):
  return pl.BlockSpec((BN, d), lambda i: (i, 0))


def _p_blk(d):
  return pl.BlockSpec((NC, BN, d), lambda i: (0, i, 0))


_DEGP_BLK = pl.BlockSpec((NC, BN, 16), lambda i: (0, i, 0))


def _full_blk(a, b):
  return pl.BlockSpec((a, b), lambda i: (0, 0))


def _mm_first(x, w, degp):
  return pl.pallas_call(
      _mm_first_body,
      grid=(GRID,),
      in_specs=[_row_blk(D_IN), _full_blk(D_IN, D_HID), _DEGP_BLK],
      out_specs=_row_blk(D_HID),
      out_shape=jax.ShapeDtypeStruct((N_NODES, D_HID), jnp.bfloat16),
  )(x, w, degp)


def _mm_mid(p, hp, b, w, degp, d_out):
  return pl.pallas_call(
      _mm_mid_body,
      grid=(GRID,),
      in_specs=[_p_blk(D_HID), _row_blk(D_HID), _full_blk(1, D_HID),
                _full_blk(D_HID, d_out), _DEGP_BLK],
      out_specs=_row_blk(d_out),
      out_shape=jax.ShapeDtypeStruct((N_NODES, d_out), jnp.bfloat16),
  )(p, hp, b, w, degp)


def _softmax_out(p, hp, b, degp):
  return pl.pallas_call(
      _soft_body,
      grid=(GRID,),
      in_specs=[_p_blk(D_OUT), _row_blk(D_OUT), _full_blk(1, D_OUT), _DEGP_BLK],
      out_specs=_row_blk(D_OUT),
      out_shape=jax.ShapeDtypeStruct((N_NODES, D_OUT), jnp.float32),
  )(p, hp, b, degp)


def kernel(x, edge_index, W1, b1, W2, b2, W3, b3, W4, b4):
  src32 = edge_index[0].astype(jnp.int32)
  dst32 = edge_index[1].astype(jnp.int32)
  src = src32.reshape(NW, N_CHUNKS, CHUNK)
  dst = dst32.reshape(NW, N_CHUNKS, CHUNK)
  dst40 = dst32.reshape(NW, DN_CHUNKS, DCHUNK)
  x = x.astype(jnp.float32)

  ones_tab = jnp.ones((N_NODES, 16), jnp.float32)
  degp = _degree(ones_tab, dst40)

  h1p = _mm_first(x, W1, degp)
  p1 = _propagate_128(h1p, src, dst)
  h2p = _mm_mid(p1, h1p, b1.reshape(1, -1), W2, degp, D_HID)
  p2 = _propagate_128(h2p, src, dst)
  h3p = _mm_mid(p2, h2p, b2.reshape(1, -1), W3, degp, D_HID)
  p3 = _propagate_128(h3p, src, dst)
  h4p = _mm_mid(p3, h3p, b3.reshape(1, -1), W4, degp, D_OUT)
  p4 = _propagate_64(h4p, src, dst)
  return _softmax_out(p4, h4p, b4.reshape(1, -1), degp)
